# baseline (speedup 1.0000x reference)
import functools

import jax
import jax.numpy as jnp
from jax import lax
from jax.experimental import pallas as pl
from jax.experimental.pallas import tpu as pltpu
from jax.experimental.pallas import tpu_sc as plsc

BATCH = 16384
EMB = 32
HID = 16
PACK = 4
LANES = EMB * PACK

NUM_CORES = 2
NUM_SUBCORES = 16
NUM_WORKERS = NUM_CORES * NUM_SUBCORES
ROWS_PER_WORKER = BATCH // NUM_WORKERS
IDX_CHUNK = 128
N_CHUNKS = ROWS_PER_WORKER // IDX_CHUNK


def _sc_gather_mul(x0_hbm, x1_hbm, ptab_hbm, itab_hbm, h_hbm,
                   idx0_v, idx1_v, prows_v, irows_v, sem):
  wid = lax.axis_index("s") * NUM_CORES + lax.axis_index("c")
  base = wid * ROWS_PER_WORKER

  pltpu.sync_copy(x0_hbm.at[pl.ds(wid * N_CHUNKS, N_CHUNKS)], idx0_v)
  pltpu.sync_copy(x1_hbm.at[pl.ds(wid * N_CHUNKS, N_CHUNKS)], idx1_v)

  copies = []
  for j in range(N_CHUNKS):
    dst = prows_v.at[pl.ds(j * IDX_CHUNK, IDX_CHUNK)]
    c = pltpu.make_async_copy(ptab_hbm.at[idx0_v.at[j]], dst, sem)
    c.start()
    copies.append(c)
  for j in range(N_CHUNKS):
    dst = irows_v.at[pl.ds(j * IDX_CHUNK, IDX_CHUNK)]
    c = pltpu.make_async_copy(itab_hbm.at[idx1_v.at[j]], dst, sem)
    c.start()
    copies.append(c)
  for c in copies:
    c.wait()

  def body(r, _):
    for half in range(2):
      sl = pl.ds(half * 16, 16)
      prows_v[r, sl] = prows_v[r, sl] * irows_v[r, sl]
    return _

  lax.fori_loop(0, ROWS_PER_WORKER, body, 0, unroll=8)

  pltpu.sync_copy(prows_v, h_hbm.at[pl.ds(base, ROWS_PER_WORKER)])


def _tc_bn_mlp(h_ref, gamma_ref, beta_ref, w1_ref, b1_ref, w2_ref, b2_ref,
               out_ref):
  h = h_ref[...]
  s = jnp.sum(h, axis=0, keepdims=True)
  sq = jnp.sum(h * h, axis=0, keepdims=True)
  s32 = (s[:, 0:32] + s[:, 32:64]) + (s[:, 64:96] + s[:, 96:128])
  sq32 = (sq[:, 0:32] + sq[:, 32:64]) + (sq[:, 64:96] + sq[:, 96:128])
  mean = s32 * (1.0 / BATCH)
  var = sq32 * (1.0 / BATCH) - mean * mean
  a32 = gamma_ref[...] * lax.rsqrt(var + 1e-5)
  c32 = beta_ref[...] - mean * a32
  a = jnp.concatenate([a32, a32, a32, a32], axis=1)
  c = jnp.concatenate([c32, c32, c32, c32], axis=1)
  hn = h * a + c

  z1 = jnp.dot(hn, w1_ref[...], preferred_element_type=jnp.float32)
  z1 = z1 + b1_ref[...]
  scale = 1.0507009873554804934193349852946
  alpha = 1.6732632423543772848170429916717
  z1 = scale * jnp.where(z1 > 0, z1, alpha * (jnp.exp(z1) - 1.0))
  z2 = jnp.dot(z1, w2_ref[...], preferred_element_type=jnp.float32)
  z2 = z2 + b2_ref[...]
  out_ref[...] = jax.nn.sigmoid(z2)


@jax.jit
def kernel(x, playlist_emb, item_emb, bn_gamma, bn_beta, W1, b1, W2, b2):
  x0 = x[:, 0].astype(jnp.int32).reshape(IDX_CHUNK, IDX_CHUNK)
  x1 = x[:, 1].astype(jnp.int32).reshape(IDX_CHUNK, IDX_CHUNK)

  mesh = plsc.VectorSubcoreMesh(core_axis_name="c", subcore_axis_name="s")
  gather_mul = pl.kernel(
      _sc_gather_mul,
      out_type=jax.ShapeDtypeStruct((BATCH, EMB), jnp.float32),
      mesh=mesh,
      compiler_params=pltpu.CompilerParams(use_tc_tiling_on_sc=False),
      scratch_types=[
          pltpu.VMEM((N_CHUNKS, IDX_CHUNK), jnp.int32),
          pltpu.VMEM((N_CHUNKS, IDX_CHUNK), jnp.int32),
          pltpu.VMEM((ROWS_PER_WORKER, EMB), jnp.float32),
          pltpu.VMEM((ROWS_PER_WORKER, EMB), jnp.float32),
          pltpu.SemaphoreType.DMA,
      ],
  )
  h = gather_mul(x0, x1, playlist_emb, item_emb)

  h2 = h.reshape(BATCH // PACK, LANES)

  eye = jnp.eye(PACK, dtype=jnp.float32)
  w1big = jnp.kron(eye, W1.T)
  b1big = jnp.tile(b1, PACK).reshape(1, PACK * HID)
  w2big = jnp.kron(eye, W2.T)
  b2big = jnp.tile(b2, PACK).reshape(1, PACK)

  out = pl.pallas_call(
      _tc_bn_mlp,
      out_shape=jax.ShapeDtypeStruct((BATCH // PACK, PACK), jnp.float32),
  )(h2, bn_gamma.reshape(1, EMB), bn_beta.reshape(1, EMB),
    w1big, b1big, w2big, b2big)

  return out.reshape(BATCH, 1)
